# Initial kernel scaffold; baseline (speedup 1.0000x reference)
import jax
import jax.numpy as jnp
from jax.experimental import pallas as pl


def kernel(x, edge_index, edge_weights, batch, W1, b1, W2, b2, Wc, bc):
    raise NotImplementedError("write your pallas kernel here")


# speedup vs baseline: 7.5763x; 7.5763x over previous
import functools

import jax
import jax.numpy as jnp
from jax import lax
from jax.experimental import pallas as pl
from jax.experimental.pallas import tpu as pltpu
from jax.experimental.pallas import tpu_sc as plsc

N = 10000
E = 320000
D = 128
G = 64
C = 16

NC = 2
NS = 16
NW = NC * NS

EPT_ROWS = 80
ROWS_TOT = NW * EPT_ROWS
E_PAD = ROWS_TOT * 128
SEG = 16
NSEG = EPT_ROWS // SEG

NPAD = 10240
ROWS_PER_TILE = NPAD // NS

BLK = 1000
NBLK = N // BLK


def _deg_body(dst_hbm, ew_hbm, zeros_hbm, out_hbm,
              dstb, ewb, rows, acc_sh):
    cid = lax.axis_index("c")
    sid = lax.axis_index("s")
    wid = cid * NS + sid
    pltpu.sync_copy(zeros_hbm.at[pl.ds(sid * ROWS_PER_TILE, ROWS_PER_TILE)],
                    acc_sh.at[pl.ds(sid * ROWS_PER_TILE, ROWS_PER_TILE)])

    z16 = jnp.zeros((16,), jnp.float32)

    def zrow(i, c):
        for cc in range(8):
            rows[i, pl.ds(cc * 16, 16)] = z16
        return c

    lax.fori_loop(0, 128, zrow, 0)
    plsc.subcore_barrier()

    row0 = wid * EPT_ROWS

    def seg_body(t, carry):
        base = row0 + t * SEG
        pltpu.sync_copy(dst_hbm.at[pl.ds(base, SEG)], dstb)
        pltpu.sync_copy(ew_hbm.at[pl.ds(base, SEG)], ewb)

        def row_body(j, c2):
            def grp_body(e16, c3):
                wv = ewb[j, pl.ds(e16 * 16, 16)]
                for kk in range(16):
                    rows[e16 * 16 + kk, pl.ds(0, 16)] = jnp.full(
                        (16,), wv[kk], dtype=jnp.float32)
                return c3

            lax.fori_loop(0, 8, grp_body, 0)
            pltpu.sync_copy(rows, acc_sh.at[dstb.at[j]], add=True)
            return c2

        return lax.fori_loop(0, SEG, row_body, carry)

    lax.fori_loop(0, NSEG, seg_body, 0)
    plsc.subcore_barrier()
    pltpu.sync_copy(acc_sh.at[pl.ds(sid * ROWS_PER_TILE, ROWS_PER_TILE)],
                    out_hbm.at[cid, pl.ds(sid * ROWS_PER_TILE, ROWS_PER_TILE)])


_deg_call = functools.partial(
    pl.kernel,
    mesh=plsc.VectorSubcoreMesh(core_axis_name="c", subcore_axis_name="s"),
    out_type=jax.ShapeDtypeStruct((NC, NPAD, D), jnp.float32),
    scratch_types=[
        pltpu.VMEM((SEG, 128), jnp.int32),
        pltpu.VMEM((SEG, 128), jnp.float32),
        pltpu.VMEM((128, D), jnp.float32),
        pltpu.VMEM_SHARED((NPAD, D), jnp.float32),
    ],
)(_deg_body)


def _msg_body(xs_hbm, src_hbm, dst_hbm, ew_hbm, zeros_hbm, out_hbm,
              srcb, dstb, ewb, rows, acc_sh, gsem):
    cid = lax.axis_index("c")
    sid = lax.axis_index("s")
    wid = cid * NS + sid

    pltpu.sync_copy(zeros_hbm.at[pl.ds(sid * ROWS_PER_TILE, ROWS_PER_TILE)],
                    acc_sh.at[pl.ds(sid * ROWS_PER_TILE, ROWS_PER_TILE)])
    plsc.subcore_barrier()

    row0 = wid * EPT_ROWS

    def seg_body(t, carry):
        base = row0 + t * SEG
        pltpu.sync_copy(src_hbm.at[pl.ds(base, SEG)], srcb)
        pltpu.sync_copy(dst_hbm.at[pl.ds(base, SEG)], dstb)
        pltpu.sync_copy(ew_hbm.at[pl.ds(base, SEG)], ewb)

        def row_body(j, c2):
            pltpu.async_copy(xs_hbm.at[srcb.at[j]], rows, gsem).wait()

            def grp_body(e16, c3):
                wv = ewb[j, pl.ds(e16 * 16, 16)]
                for kk in range(16):
                    sv = jnp.full((16,), wv[kk], dtype=jnp.float32)
                    erow = e16 * 16 + kk
                    for c in range(8):
                        rows[erow, pl.ds(c * 16, 16)] = (
                            rows[erow, pl.ds(c * 16, 16)] * sv)
                return c3

            lax.fori_loop(0, 8, grp_body, 0)
            pltpu.sync_copy(rows, acc_sh.at[dstb.at[j]], add=True)
            return c2

        return lax.fori_loop(0, SEG, row_body, carry)

    lax.fori_loop(0, NSEG, seg_body, 0)
    plsc.subcore_barrier()
    pltpu.sync_copy(acc_sh.at[pl.ds(sid * ROWS_PER_TILE, ROWS_PER_TILE)],
                    out_hbm.at[cid, pl.ds(sid * ROWS_PER_TILE, ROWS_PER_TILE)])


_msg_call = functools.partial(
    pl.kernel,
    mesh=plsc.VectorSubcoreMesh(core_axis_name="c", subcore_axis_name="s"),
    out_type=jax.ShapeDtypeStruct((NC, NPAD, D), jnp.float32),
    scratch_types=[
        pltpu.VMEM((SEG, 128), jnp.int32),
        pltpu.VMEM((SEG, 128), jnp.int32),
        pltpu.VMEM((SEG, 128), jnp.float32),
        pltpu.VMEM((128, D), jnp.float32),
        pltpu.VMEM_SHARED((NPAD, D), jnp.float32),
        pltpu.SemaphoreType.DMA,
    ],
)(_msg_body)


def _enc_body(x_ref, w_ref, b_ref, deg_ref, xs_ref, t_ref, dinv_ref):
    xw = jnp.dot(x_ref[...], w_ref[...], preferred_element_type=jnp.float32)
    deg = deg_ref[...]
    dinv = jnp.where(deg > 0.0, lax.rsqrt(deg), 0.0)
    xs_ref[...] = xw * dinv
    t_ref[...] = xw * (dinv * dinv) + b_ref[...]
    dinv_ref[...] = dinv


def _enc_call(x, w, b_row, deg_col):
    return pl.pallas_call(
        _enc_body,
        grid=(NBLK,),
        in_specs=[
            pl.BlockSpec((BLK, D), lambda i: (i, 0)),
            pl.BlockSpec((D, D), lambda i: (0, 0)),
            pl.BlockSpec((1, D), lambda i: (0, 0)),
            pl.BlockSpec((BLK, 1), lambda i: (i, 0)),
        ],
        out_specs=[
            pl.BlockSpec((BLK, D), lambda i: (i, 0)),
            pl.BlockSpec((BLK, D), lambda i: (i, 0)),
            pl.BlockSpec((BLK, 1), lambda i: (i, 0)),
        ],
        out_shape=[
            jax.ShapeDtypeStruct((N, D), jnp.float32),
            jax.ShapeDtypeStruct((N, D), jnp.float32),
            jax.ShapeDtypeStruct((N, 1), jnp.float32),
        ],
    )(x, w, b_row, deg_col)


def _mid_body(acc_ref, t_ref, dinv_ref, w_ref, b_ref, h_ref, xs_ref, t2_ref):
    dinv = dinv_ref[...]
    h = jnp.maximum((acc_ref[0] + acc_ref[1]) * dinv + t_ref[...], 0.0)
    h_ref[...] = h
    xw = jnp.dot(h, w_ref[...], preferred_element_type=jnp.float32)
    xs_ref[...] = xw * dinv
    t2_ref[...] = xw * (dinv * dinv) + b_ref[...]


def _mid_call(acc, t, dinv_col, w, b_row):
    return pl.pallas_call(
        _mid_body,
        grid=(NBLK,),
        in_specs=[
            pl.BlockSpec((NC, BLK, D), lambda i: (0, i, 0)),
            pl.BlockSpec((BLK, D), lambda i: (i, 0)),
            pl.BlockSpec((BLK, 1), lambda i: (i, 0)),
            pl.BlockSpec((D, D), lambda i: (0, 0)),
            pl.BlockSpec((1, D), lambda i: (0, 0)),
        ],
        out_specs=[
            pl.BlockSpec((BLK, D), lambda i: (i, 0)),
            pl.BlockSpec((BLK, D), lambda i: (i, 0)),
            pl.BlockSpec((BLK, D), lambda i: (i, 0)),
        ],
        out_shape=[
            jax.ShapeDtypeStruct((N, D), jnp.float32),
            jax.ShapeDtypeStruct((N, D), jnp.float32),
            jax.ShapeDtypeStruct((N, D), jnp.float32),
        ],
    )(acc, t, dinv_col, w, b_row)


def _fin_body(acc_ref, t_ref, dinv_ref, h1_ref, batch_ref, wc_ref, bc_ref,
              out_ref, sums, counts):
    i = pl.program_id(0)

    @pl.when(i == 0)
    def _():
        sums[...] = jnp.zeros_like(sums)
        counts[...] = jnp.zeros_like(counts)

    dinv = dinv_ref[...]
    h2 = jnp.maximum((acc_ref[0] + acc_ref[1]) * dinv + t_ref[...], 0.0)
    b = batch_ref[0]
    ids = lax.broadcasted_iota(jnp.int32, (G, BLK), 0)
    mask = (b == ids).astype(jnp.float32)
    sums[:, 0:D] = sums[:, 0:D] + jnp.dot(
        mask, h1_ref[...], preferred_element_type=jnp.float32)
    sums[:, D:2 * D] = sums[:, D:2 * D] + jnp.dot(
        mask, h2, preferred_element_type=jnp.float32)
    counts[...] = counts[...] + jnp.sum(mask, axis=1, keepdims=True)

    @pl.when(i == pl.num_programs(0) - 1)
    def _():
        cnt = counts[:, 0:1]
        gemb = sums[...] / jnp.maximum(cnt, 1.0)
        out_ref[...] = jnp.dot(
            gemb, wc_ref[...], preferred_element_type=jnp.float32) + bc_ref[...]


def _fin_call(acc, t, dinv_col, h1, batch3, wc_pad, bc_pad):
    return pl.pallas_call(
        _fin_body,
        grid=(NBLK,),
        in_specs=[
            pl.BlockSpec((NC, BLK, D), lambda i: (0, i, 0)),
            pl.BlockSpec((BLK, D), lambda i: (i, 0)),
            pl.BlockSpec((BLK, 1), lambda i: (i, 0)),
            pl.BlockSpec((BLK, D), lambda i: (i, 0)),
            pl.BlockSpec((1, 1, BLK), lambda i: (i, 0, 0)),
            pl.BlockSpec((2 * D, 128), lambda i: (0, 0)),
            pl.BlockSpec((1, 128), lambda i: (0, 0)),
        ],
        out_specs=pl.BlockSpec((G, 128), lambda i: (0, 0)),
        out_shape=jax.ShapeDtypeStruct((G, 128), jnp.float32),
        scratch_shapes=[
            pltpu.VMEM((G, 2 * D), jnp.float32),
            pltpu.VMEM((G, 128), jnp.float32),
        ],
    )(acc, t, dinv_col, h1, batch3, wc_pad, bc_pad)


def kernel(x, edge_index, edge_weights, batch, W1, b1, W2, b2, Wc, bc):
    src = edge_index[0].astype(jnp.int32)
    dst = edge_index[1].astype(jnp.int32)
    ew = edge_weights.astype(jnp.float32)

    pad = E_PAD - E
    zi = jnp.zeros((pad,), jnp.int32)
    src_p = jnp.concatenate([src, zi]).reshape(ROWS_TOT, 128)
    dst_p = jnp.concatenate([dst, zi]).reshape(ROWS_TOT, 128)
    ew_p = jnp.concatenate([ew, jnp.zeros((pad,), jnp.float32)]).reshape(
        ROWS_TOT, 128)

    zeros_acc = jnp.zeros((NPAD, D), jnp.float32)

    degp = _deg_call(dst_p, ew_p, zeros_acc)
    deg_col = (degp[0, :N, 0] + degp[1, :N, 0] + 1.0).reshape(N, 1)

    b1r = b1.reshape(1, D)
    b2r = b2.reshape(1, D)

    xs1, t1, dinv_col = _enc_call(x, W1, b1r, deg_col)

    acc1 = _msg_call(xs1, src_p, dst_p, ew_p, zeros_acc)[:, :N, :]
    h1, xs2, t2 = _mid_call(acc1, t1, dinv_col, W2, b2r)
    acc2 = _msg_call(xs2, src_p, dst_p, ew_p, zeros_acc)[:, :N, :]

    batch3 = batch.astype(jnp.int32).reshape(NBLK, 1, BLK)
    wc_pad = jnp.zeros((2 * D, 128), jnp.float32).at[:, :C].set(Wc)
    bc_pad = jnp.zeros((1, 128), jnp.float32).at[:, :C].set(bc.reshape(1, C))

    logits_pad = _fin_call(acc2, t2, dinv_col, h1, batch3, wc_pad, bc_pad)
    return logits_pad[:, :C]


# speedup vs baseline: 9.4946x; 1.2532x over previous
import functools

import jax
import jax.numpy as jnp
from jax import lax
from jax.experimental import pallas as pl
from jax.experimental.pallas import tpu as pltpu
from jax.experimental.pallas import tpu_sc as plsc

N = 10000
E = 320000
D = 128
G = 64
C = 16

NC = 2
NS = 16
NW = NC * NS

EPT_ROWS = 80
ROWS_TOT = NW * EPT_ROWS
E_PAD = ROWS_TOT * 128
SEG = 16
NSEG = EPT_ROWS // SEG

NPAD = 10240
ROWS_PER_TILE = NPAD // NS

BLK = 1000
NBLK = N // BLK


def _deg_body(dst_hbm, ew_hbm, zeros_hbm, out_hbm,
              dstb, ewb, rows, acc_sh):
    cid = lax.axis_index("c")
    sid = lax.axis_index("s")
    wid = cid * NS + sid
    pltpu.sync_copy(zeros_hbm.at[pl.ds(sid * ROWS_PER_TILE, ROWS_PER_TILE)],
                    acc_sh.at[pl.ds(sid * ROWS_PER_TILE, ROWS_PER_TILE)])

    z16 = jnp.zeros((16,), jnp.float32)

    def zrow(i, c):
        for cc in range(8):
            rows[i, pl.ds(cc * 16, 16)] = z16
        return c

    lax.fori_loop(0, 128, zrow, 0)
    plsc.subcore_barrier()

    row0 = wid * EPT_ROWS

    def seg_body(t, carry):
        base = row0 + t * SEG
        pltpu.sync_copy(dst_hbm.at[pl.ds(base, SEG)], dstb)
        pltpu.sync_copy(ew_hbm.at[pl.ds(base, SEG)], ewb)

        def row_body(j, c2):
            def grp_body(e16, c3):
                wv = ewb[j, pl.ds(e16 * 16, 16)]
                for kk in range(16):
                    rows[e16 * 16 + kk, pl.ds(0, 16)] = jnp.full(
                        (16,), wv[kk], dtype=jnp.float32)
                return c3

            lax.fori_loop(0, 8, grp_body, 0)
            pltpu.sync_copy(rows, acc_sh.at[dstb.at[j]], add=True)
            return c2

        return lax.fori_loop(0, SEG, row_body, carry)

    lax.fori_loop(0, NSEG, seg_body, 0)
    plsc.subcore_barrier()
    pltpu.sync_copy(acc_sh.at[pl.ds(sid * ROWS_PER_TILE, ROWS_PER_TILE)],
                    out_hbm.at[cid, pl.ds(sid * ROWS_PER_TILE, ROWS_PER_TILE)])


_deg_call = functools.partial(
    pl.kernel,
    mesh=plsc.VectorSubcoreMesh(core_axis_name="c", subcore_axis_name="s"),
    out_type=jax.ShapeDtypeStruct((NC, NPAD, D), jnp.float32),
    scratch_types=[
        pltpu.VMEM((SEG, 128), jnp.int32),
        pltpu.VMEM((SEG, 128), jnp.float32),
        pltpu.VMEM((128, D), jnp.float32),
        pltpu.VMEM_SHARED((NPAD, D), jnp.float32),
    ],
)(_deg_body)


def _scale_rows(rows, ewb, g):
    def grp_body(e16, c3):
        wv = ewb[g, pl.ds(e16 * 16, 16)]
        for kk in range(16):
            sv = jnp.full((16,), wv[kk], dtype=jnp.float32)
            erow = e16 * 16 + kk
            for c in range(8):
                rows[erow, pl.ds(c * 16, 16)] = (
                    rows[erow, pl.ds(c * 16, 16)] * sv)
        return c3

    lax.fori_loop(0, 8, grp_body, 0)


MSG_RTOT = ROWS_TOT // NS
MSG_R0 = 112
MSG_R1 = MSG_RTOT - MSG_R0


def _msg_body(xs_hbm, src_hbm, dst_hbm, ew_hbm, zeros_hbm, out_hbm,
              srcb, dstb, ewb, rowsA, rowsB, acc_sh,
              gsemA, gsemA2, gsemB, gsemB2):
    cid = lax.axis_index("c")
    sid = lax.axis_index("s")

    pltpu.sync_copy(zeros_hbm.at[pl.ds(sid * ROWS_PER_TILE, ROWS_PER_TILE)],
                    acc_sh.at[pl.ds(sid * ROWS_PER_TILE, ROWS_PER_TILE)])
    plsc.subcore_barrier()

    row0 = jnp.where(cid == 0, sid * MSG_R0, MSG_R0 * NS + sid * MSG_R1)
    nseg = jnp.where(cid == 0, MSG_R0 // SEG, MSG_R1 // SEG)

    def seg_body(t, carry):
        base = row0 + t * SEG
        pltpu.sync_copy(src_hbm.at[pl.ds(base, SEG)], srcb)
        pltpu.sync_copy(dst_hbm.at[pl.ds(base, SEG)], dstb)
        pltpu.sync_copy(ew_hbm.at[pl.ds(base, SEG)], ewb)
        def start_gather(j, rows, semL, semH):
            pltpu.async_copy(xs_hbm.at[srcb.at[j, pl.ds(0, 64)]],
                             rows.at[pl.ds(0, 64)], semL)
            pltpu.async_copy(xs_hbm.at[srcb.at[j, pl.ds(64, 64)]],
                             rows.at[pl.ds(64, 64)], semH)

        def wait_gather(j, rows, semL, semH):
            pltpu.make_async_copy(xs_hbm.at[srcb.at[j, pl.ds(0, 64)]],
                                  rows.at[pl.ds(0, 64)], semL).wait()
            pltpu.make_async_copy(xs_hbm.at[srcb.at[j, pl.ds(64, 64)]],
                                  rows.at[pl.ds(64, 64)], semH).wait()

        start_gather(0, rowsA, gsemA, gsemA2)

        def pair_body(qq, c2):
            j0 = 2 * qq
            j1 = j0 + 1
            start_gather(j1, rowsB, gsemB, gsemB2)
            wait_gather(j0, rowsA, gsemA, gsemA2)
            _scale_rows(rowsA, ewb, j0)
            pltpu.sync_copy(rowsA, acc_sh.at[dstb.at[j0]], add=True)

            @pl.when(j0 + 2 < SEG)
            def _():
                start_gather(j0 + 2, rowsA, gsemA, gsemA2)

            wait_gather(j1, rowsB, gsemB, gsemB2)
            _scale_rows(rowsB, ewb, j1)
            pltpu.sync_copy(rowsB, acc_sh.at[dstb.at[j1]], add=True)
            return c2

        return lax.fori_loop(0, SEG // 2, pair_body, carry)

    lax.fori_loop(0, nseg, seg_body, 0)
    plsc.subcore_barrier()
    pltpu.sync_copy(acc_sh.at[pl.ds(sid * ROWS_PER_TILE, ROWS_PER_TILE)],
                    out_hbm.at[cid, pl.ds(sid * ROWS_PER_TILE, ROWS_PER_TILE)])


_msg_call = functools.partial(
    pl.kernel,
    mesh=plsc.VectorSubcoreMesh(core_axis_name="c", subcore_axis_name="s"),
    out_type=jax.ShapeDtypeStruct((NC, NPAD, D), jnp.float32),
    scratch_types=[
        pltpu.VMEM((SEG, 128), jnp.int32),
        pltpu.VMEM((SEG, 128), jnp.int32),
        pltpu.VMEM((SEG, 128), jnp.float32),
        pltpu.VMEM((128, D), jnp.float32),
        pltpu.VMEM((128, D), jnp.float32),
        pltpu.VMEM_SHARED((NPAD, D), jnp.float32),
        pltpu.SemaphoreType.DMA,
        pltpu.SemaphoreType.DMA,
        pltpu.SemaphoreType.DMA,
        pltpu.SemaphoreType.DMA,
    ],
)(_msg_body)


def _enc_body(x_ref, w_ref, b_ref, deg_ref, xs_ref, t_ref, dinv_ref):
    xw = jnp.dot(x_ref[...], w_ref[...], preferred_element_type=jnp.float32)
    deg = deg_ref[...]
    dinv = jnp.where(deg > 0.0, lax.rsqrt(deg), 0.0)
    xs_ref[...] = xw * dinv
    t_ref[...] = xw * (dinv * dinv) + b_ref[...]
    dinv_ref[...] = dinv


def _enc_call(x, w, b_row, deg_col):
    return pl.pallas_call(
        _enc_body,
        grid=(NBLK,),
        in_specs=[
            pl.BlockSpec((BLK, D), lambda i: (i, 0)),
            pl.BlockSpec((D, D), lambda i: (0, 0)),
            pl.BlockSpec((1, D), lambda i: (0, 0)),
            pl.BlockSpec((BLK, 1), lambda i: (i, 0)),
        ],
        out_specs=[
            pl.BlockSpec((BLK, D), lambda i: (i, 0)),
            pl.BlockSpec((BLK, D), lambda i: (i, 0)),
            pl.BlockSpec((BLK, 1), lambda i: (i, 0)),
        ],
        out_shape=[
            jax.ShapeDtypeStruct((N, D), jnp.float32),
            jax.ShapeDtypeStruct((N, D), jnp.float32),
            jax.ShapeDtypeStruct((N, 1), jnp.float32),
        ],
    )(x, w, b_row, deg_col)


def _mid_body(acc_ref, t_ref, dinv_ref, w_ref, b_ref, h_ref, xs_ref, t2_ref):
    dinv = dinv_ref[...]
    h = jnp.maximum((acc_ref[0] + acc_ref[1]) * dinv + t_ref[...], 0.0)
    h_ref[...] = h
    xw = jnp.dot(h, w_ref[...], preferred_element_type=jnp.float32)
    xs_ref[...] = xw * dinv
    t2_ref[...] = xw * (dinv * dinv) + b_ref[...]


def _mid_call(acc, t, dinv_col, w, b_row):
    return pl.pallas_call(
        _mid_body,
        grid=(NBLK,),
        in_specs=[
            pl.BlockSpec((NC, BLK, D), lambda i: (0, i, 0)),
            pl.BlockSpec((BLK, D), lambda i: (i, 0)),
            pl.BlockSpec((BLK, 1), lambda i: (i, 0)),
            pl.BlockSpec((D, D), lambda i: (0, 0)),
            pl.BlockSpec((1, D), lambda i: (0, 0)),
        ],
        out_specs=[
            pl.BlockSpec((BLK, D), lambda i: (i, 0)),
            pl.BlockSpec((BLK, D), lambda i: (i, 0)),
            pl.BlockSpec((BLK, D), lambda i: (i, 0)),
        ],
        out_shape=[
            jax.ShapeDtypeStruct((N, D), jnp.float32),
            jax.ShapeDtypeStruct((N, D), jnp.float32),
            jax.ShapeDtypeStruct((N, D), jnp.float32),
        ],
    )(acc, t, dinv_col, w, b_row)


def _fin_body(acc_ref, t_ref, dinv_ref, h1_ref, batch_ref, wc_ref, bc_ref,
              out_ref, sums, counts):
    i = pl.program_id(0)

    @pl.when(i == 0)
    def _():
        sums[...] = jnp.zeros_like(sums)
        counts[...] = jnp.zeros_like(counts)

    dinv = dinv_ref[...]
    h2 = jnp.maximum((acc_ref[0] + acc_ref[1]) * dinv + t_ref[...], 0.0)
    b = batch_ref[0]
    ids = lax.broadcasted_iota(jnp.int32, (G, BLK), 0)
    mask = (b == ids).astype(jnp.float32)
    sums[:, 0:D] = sums[:, 0:D] + jnp.dot(
        mask, h1_ref[...], preferred_element_type=jnp.float32)
    sums[:, D:2 * D] = sums[:, D:2 * D] + jnp.dot(
        mask, h2, preferred_element_type=jnp.float32)
    counts[...] = counts[...] + jnp.sum(mask, axis=1, keepdims=True)

    @pl.when(i == pl.num_programs(0) - 1)
    def _():
        cnt = counts[:, 0:1]
        gemb = sums[...] / jnp.maximum(cnt, 1.0)
        out_ref[...] = jnp.dot(
            gemb, wc_ref[...], preferred_element_type=jnp.float32) + bc_ref[...]


def _fin_call(acc, t, dinv_col, h1, batch3, wc_pad, bc_pad):
    return pl.pallas_call(
        _fin_body,
        grid=(NBLK,),
        in_specs=[
            pl.BlockSpec((NC, BLK, D), lambda i: (0, i, 0)),
            pl.BlockSpec((BLK, D), lambda i: (i, 0)),
            pl.BlockSpec((BLK, 1), lambda i: (i, 0)),
            pl.BlockSpec((BLK, D), lambda i: (i, 0)),
            pl.BlockSpec((1, 1, BLK), lambda i: (i, 0, 0)),
            pl.BlockSpec((2 * D, 128), lambda i: (0, 0)),
            pl.BlockSpec((1, 128), lambda i: (0, 0)),
        ],
        out_specs=pl.BlockSpec((G, 128), lambda i: (0, 0)),
        out_shape=jax.ShapeDtypeStruct((G, 128), jnp.float32),
        scratch_shapes=[
            pltpu.VMEM((G, 2 * D), jnp.float32),
            pltpu.VMEM((G, 128), jnp.float32),
        ],
    )(acc, t, dinv_col, h1, batch3, wc_pad, bc_pad)


def kernel(x, edge_index, edge_weights, batch, W1, b1, W2, b2, Wc, bc):
    src = edge_index[0].astype(jnp.int32)
    dst = edge_index[1].astype(jnp.int32)
    ew = edge_weights.astype(jnp.float32)

    pad = E_PAD - E
    zi = jnp.zeros((pad,), jnp.int32)
    src_p = jnp.concatenate([src, zi]).reshape(ROWS_TOT, 128)
    dst_p = jnp.concatenate([dst, zi]).reshape(ROWS_TOT, 128)
    ew_p = jnp.concatenate([ew, jnp.zeros((pad,), jnp.float32)]).reshape(
        ROWS_TOT, 128)

    zeros_acc = jnp.zeros((NPAD, D), jnp.float32)

    degp = _deg_call(dst_p, ew_p, zeros_acc)
    deg_col = (degp[0, :N, 0] + degp[1, :N, 0] + 1.0).reshape(N, 1)

    b1r = b1.reshape(1, D)
    b2r = b2.reshape(1, D)

    xs1, t1, dinv_col = _enc_call(x, W1, b1r, deg_col)

    acc1 = _msg_call(xs1, src_p, dst_p, ew_p, zeros_acc)[:, :N, :]
    h1, xs2, t2 = _mid_call(acc1, t1, dinv_col, W2, b2r)
    acc2 = _msg_call(xs2, src_p, dst_p, ew_p, zeros_acc)[:, :N, :]

    batch3 = batch.astype(jnp.int32).reshape(NBLK, 1, BLK)
    wc_pad = jnp.zeros((2 * D, 128), jnp.float32).at[:, :C].set(Wc)
    bc_pad = jnp.zeros((1, 128), jnp.float32).at[:, :C].set(bc.reshape(1, C))

    logits_pad = _fin_call(acc2, t2, dinv_col, h1, batch3, wc_pad, bc_pad)
    return logits_pad[:, :C]


# speedup vs baseline: 9.8595x; 1.0384x over previous
import functools

import jax
import jax.numpy as jnp
from jax import lax
from jax.experimental import pallas as pl
from jax.experimental.pallas import tpu as pltpu
from jax.experimental.pallas import tpu_sc as plsc

N = 10000
E = 320000
D = 128
G = 64
C = 16

NC = 2
NS = 16
NW = NC * NS

EPT_ROWS = 80
ROWS_TOT = NW * EPT_ROWS
E_PAD = ROWS_TOT * 128
SEG = 16
NSEG = EPT_ROWS // SEG

NPAD = 10240
ROWS_PER_TILE = NPAD // NS

BLK = 1000
NBLK = N // BLK


def _deg_body(dst_hbm, ew_hbm, zeros_hbm, out_hbm,
              dstb, ewb, rows, acc_sh):
    cid = lax.axis_index("c")
    sid = lax.axis_index("s")
    wid = cid * NS + sid
    pltpu.sync_copy(zeros_hbm.at[pl.ds(sid * ROWS_PER_TILE, ROWS_PER_TILE)],
                    acc_sh.at[pl.ds(sid * ROWS_PER_TILE, ROWS_PER_TILE)])

    z16 = jnp.zeros((16,), jnp.float32)

    def zrow(i, c):
        for cc in range(8):
            rows[i, pl.ds(cc * 16, 16)] = z16
        return c

    lax.fori_loop(0, 128, zrow, 0)
    plsc.subcore_barrier()

    row0 = wid * EPT_ROWS

    def seg_body(t, carry):
        base = row0 + t * SEG
        pltpu.sync_copy(dst_hbm.at[pl.ds(base, SEG)], dstb)
        pltpu.sync_copy(ew_hbm.at[pl.ds(base, SEG)], ewb)

        def row_body(j, c2):
            def grp_body(e16, c3):
                wv = ewb[j, pl.ds(e16 * 16, 16)]
                for kk in range(16):
                    rows[e16 * 16 + kk, pl.ds(0, 16)] = jnp.full(
                        (16,), wv[kk], dtype=jnp.float32)
                return c3

            lax.fori_loop(0, 8, grp_body, 0)
            pltpu.sync_copy(rows, acc_sh.at[dstb.at[j]], add=True)
            return c2

        return lax.fori_loop(0, SEG, row_body, carry)

    lax.fori_loop(0, NSEG, seg_body, 0)
    plsc.subcore_barrier()
    pltpu.sync_copy(acc_sh.at[pl.ds(sid * ROWS_PER_TILE, ROWS_PER_TILE)],
                    out_hbm.at[cid, pl.ds(sid * ROWS_PER_TILE, ROWS_PER_TILE)])


_deg_call = functools.partial(
    pl.kernel,
    mesh=plsc.VectorSubcoreMesh(core_axis_name="c", subcore_axis_name="s"),
    out_type=jax.ShapeDtypeStruct((NC, NPAD, D), jnp.float32),
    scratch_types=[
        pltpu.VMEM((SEG, 128), jnp.int32),
        pltpu.VMEM((SEG, 128), jnp.float32),
        pltpu.VMEM((128, D), jnp.float32),
        pltpu.VMEM_SHARED((NPAD, D), jnp.float32),
    ],
)(_deg_body)


def _scale_rows(rows, ewb, g):
    def grp_body(e16, c3):
        wv = ewb[g, pl.ds(e16 * 16, 16)]
        for kk in range(16):
            sv = jnp.full((16,), wv[kk], dtype=jnp.float32)
            erow = e16 * 16 + kk
            for c in range(8):
                rows[erow, pl.ds(c * 16, 16)] = (
                    rows[erow, pl.ds(c * 16, 16)] * sv)
        return c3

    lax.fori_loop(0, 8, grp_body, 0)


MSG_RTOT = ROWS_TOT // NS
MSG_R0 = 128
MSG_R1 = MSG_RTOT - MSG_R0


def _msg_body(xs_hbm, src_hbm, dst_hbm, ew_hbm, zeros_hbm, out_hbm,
              srcb, dstb, ewb, rowsA, rowsB, acc_sh,
              gsemA, gsemA2, gsemB, gsemB2):
    cid = lax.axis_index("c")
    sid = lax.axis_index("s")

    pltpu.sync_copy(zeros_hbm.at[pl.ds(sid * ROWS_PER_TILE, ROWS_PER_TILE)],
                    acc_sh.at[pl.ds(sid * ROWS_PER_TILE, ROWS_PER_TILE)])
    plsc.subcore_barrier()

    row0 = jnp.where(cid == 0, sid * MSG_R0, MSG_R0 * NS + sid * MSG_R1)
    nseg = jnp.where(cid == 0, MSG_R0 // SEG, MSG_R1 // SEG)

    def seg_body(t, carry):
        base = row0 + t * SEG
        pltpu.sync_copy(src_hbm.at[pl.ds(base, SEG)], srcb)
        pltpu.sync_copy(dst_hbm.at[pl.ds(base, SEG)], dstb)
        pltpu.sync_copy(ew_hbm.at[pl.ds(base, SEG)], ewb)
        def start_gather(j, rows, semL, semH):
            pltpu.async_copy(xs_hbm.at[srcb.at[j, pl.ds(0, 64)]],
                             rows.at[pl.ds(0, 64)], semL)
            pltpu.async_copy(xs_hbm.at[srcb.at[j, pl.ds(64, 64)]],
                             rows.at[pl.ds(64, 64)], semH)

        def wait_gather(j, rows, semL, semH):
            pltpu.make_async_copy(xs_hbm.at[srcb.at[j, pl.ds(0, 64)]],
                                  rows.at[pl.ds(0, 64)], semL).wait()
            pltpu.make_async_copy(xs_hbm.at[srcb.at[j, pl.ds(64, 64)]],
                                  rows.at[pl.ds(64, 64)], semH).wait()

        start_gather(0, rowsA, gsemA, gsemA2)

        def pair_body(qq, c2):
            j0 = 2 * qq
            j1 = j0 + 1
            start_gather(j1, rowsB, gsemB, gsemB2)
            wait_gather(j0, rowsA, gsemA, gsemA2)
            _scale_rows(rowsA, ewb, j0)
            pltpu.sync_copy(rowsA, acc_sh.at[dstb.at[j0]], add=True)

            @pl.when(j0 + 2 < SEG)
            def _():
                start_gather(j0 + 2, rowsA, gsemA, gsemA2)

            wait_gather(j1, rowsB, gsemB, gsemB2)
            _scale_rows(rowsB, ewb, j1)
            pltpu.sync_copy(rowsB, acc_sh.at[dstb.at[j1]], add=True)
            return c2

        return lax.fori_loop(0, SEG // 2, pair_body, carry)

    lax.fori_loop(0, nseg, seg_body, 0)
    plsc.subcore_barrier()
    pltpu.sync_copy(acc_sh.at[pl.ds(sid * ROWS_PER_TILE, ROWS_PER_TILE)],
                    out_hbm.at[cid, pl.ds(sid * ROWS_PER_TILE, ROWS_PER_TILE)])


_msg_call = functools.partial(
    pl.kernel,
    mesh=plsc.VectorSubcoreMesh(core_axis_name="c", subcore_axis_name="s"),
    out_type=jax.ShapeDtypeStruct((NC, NPAD, D), jnp.float32),
    scratch_types=[
        pltpu.VMEM((SEG, 128), jnp.int32),
        pltpu.VMEM((SEG, 128), jnp.int32),
        pltpu.VMEM((SEG, 128), jnp.float32),
        pltpu.VMEM((128, D), jnp.float32),
        pltpu.VMEM((128, D), jnp.float32),
        pltpu.VMEM_SHARED((NPAD, D), jnp.float32),
        pltpu.SemaphoreType.DMA,
        pltpu.SemaphoreType.DMA,
        pltpu.SemaphoreType.DMA,
        pltpu.SemaphoreType.DMA,
    ],
)(_msg_body)


def _enc_body(x_ref, w_ref, b_ref, deg_ref, xs_ref, t_ref, dinv_ref):
    xw = jnp.dot(x_ref[...], w_ref[...], preferred_element_type=jnp.float32)
    deg = deg_ref[...]
    dinv = jnp.where(deg > 0.0, lax.rsqrt(deg), 0.0)
    xs_ref[...] = xw * dinv
    t_ref[...] = xw * (dinv * dinv) + b_ref[...]
    dinv_ref[...] = dinv


def _enc_call(x, w, b_row, deg_col):
    return pl.pallas_call(
        _enc_body,
        grid=(NBLK,),
        in_specs=[
            pl.BlockSpec((BLK, D), lambda i: (i, 0)),
            pl.BlockSpec((D, D), lambda i: (0, 0)),
            pl.BlockSpec((1, D), lambda i: (0, 0)),
            pl.BlockSpec((BLK, 1), lambda i: (i, 0)),
        ],
        out_specs=[
            pl.BlockSpec((BLK, D), lambda i: (i, 0)),
            pl.BlockSpec((BLK, D), lambda i: (i, 0)),
            pl.BlockSpec((BLK, 1), lambda i: (i, 0)),
        ],
        out_shape=[
            jax.ShapeDtypeStruct((N, D), jnp.float32),
            jax.ShapeDtypeStruct((N, D), jnp.float32),
            jax.ShapeDtypeStruct((N, 1), jnp.float32),
        ],
    )(x, w, b_row, deg_col)


def _mid_body(acc_ref, t_ref, dinv_ref, w_ref, b_ref, h_ref, xs_ref, t2_ref):
    dinv = dinv_ref[...]
    h = jnp.maximum((acc_ref[0] + acc_ref[1]) * dinv + t_ref[...], 0.0)
    h_ref[...] = h
    xw = jnp.dot(h, w_ref[...], preferred_element_type=jnp.float32)
    xs_ref[...] = xw * dinv
    t2_ref[...] = xw * (dinv * dinv) + b_ref[...]


def _mid_call(acc, t, dinv_col, w, b_row):
    return pl.pallas_call(
        _mid_body,
        grid=(NBLK,),
        in_specs=[
            pl.BlockSpec((NC, BLK, D), lambda i: (0, i, 0)),
            pl.BlockSpec((BLK, D), lambda i: (i, 0)),
            pl.BlockSpec((BLK, 1), lambda i: (i, 0)),
            pl.BlockSpec((D, D), lambda i: (0, 0)),
            pl.BlockSpec((1, D), lambda i: (0, 0)),
        ],
        out_specs=[
            pl.BlockSpec((BLK, D), lambda i: (i, 0)),
            pl.BlockSpec((BLK, D), lambda i: (i, 0)),
            pl.BlockSpec((BLK, D), lambda i: (i, 0)),
        ],
        out_shape=[
            jax.ShapeDtypeStruct((N, D), jnp.float32),
            jax.ShapeDtypeStruct((N, D), jnp.float32),
            jax.ShapeDtypeStruct((N, D), jnp.float32),
        ],
    )(acc, t, dinv_col, w, b_row)


def _fin_body(acc_ref, t_ref, dinv_ref, h1_ref, batch_ref, wc_ref, bc_ref,
              out_ref, sums, counts):
    i = pl.program_id(0)

    @pl.when(i == 0)
    def _():
        sums[...] = jnp.zeros_like(sums)
        counts[...] = jnp.zeros_like(counts)

    dinv = dinv_ref[...]
    h2 = jnp.maximum((acc_ref[0] + acc_ref[1]) * dinv + t_ref[...], 0.0)
    b = batch_ref[0]
    ids = lax.broadcasted_iota(jnp.int32, (G, BLK), 0)
    mask = (b == ids).astype(jnp.float32)
    sums[:, 0:D] = sums[:, 0:D] + jnp.dot(
        mask, h1_ref[...], preferred_element_type=jnp.float32)
    sums[:, D:2 * D] = sums[:, D:2 * D] + jnp.dot(
        mask, h2, preferred_element_type=jnp.float32)
    counts[...] = counts[...] + jnp.sum(mask, axis=1, keepdims=True)

    @pl.when(i == pl.num_programs(0) - 1)
    def _():
        cnt = counts[:, 0:1]
        gemb = sums[...] / jnp.maximum(cnt, 1.0)
        out_ref[...] = jnp.dot(
            gemb, wc_ref[...], preferred_element_type=jnp.float32) + bc_ref[...]


def _fin_call(acc, t, dinv_col, h1, batch3, wc_pad, bc_pad):
    return pl.pallas_call(
        _fin_body,
        grid=(NBLK,),
        in_specs=[
            pl.BlockSpec((NC, BLK, D), lambda i: (0, i, 0)),
            pl.BlockSpec((BLK, D), lambda i: (i, 0)),
            pl.BlockSpec((BLK, 1), lambda i: (i, 0)),
            pl.BlockSpec((BLK, D), lambda i: (i, 0)),
            pl.BlockSpec((1, 1, BLK), lambda i: (i, 0, 0)),
            pl.BlockSpec((2 * D, 128), lambda i: (0, 0)),
            pl.BlockSpec((1, 128), lambda i: (0, 0)),
        ],
        out_specs=pl.BlockSpec((G, 128), lambda i: (0, 0)),
        out_shape=jax.ShapeDtypeStruct((G, 128), jnp.float32),
        scratch_shapes=[
            pltpu.VMEM((G, 2 * D), jnp.float32),
            pltpu.VMEM((G, 128), jnp.float32),
        ],
    )(acc, t, dinv_col, h1, batch3, wc_pad, bc_pad)


def kernel(x, edge_index, edge_weights, batch, W1, b1, W2, b2, Wc, bc):
    src = edge_index[0].astype(jnp.int32)
    dst = edge_index[1].astype(jnp.int32)
    ew = edge_weights.astype(jnp.float32)

    pad = E_PAD - E
    zi = jnp.zeros((pad,), jnp.int32)
    src_p = jnp.concatenate([src, zi]).reshape(ROWS_TOT, 128)
    dst_p = jnp.concatenate([dst, zi]).reshape(ROWS_TOT, 128)
    ew_p = jnp.concatenate([ew, jnp.zeros((pad,), jnp.float32)]).reshape(
        ROWS_TOT, 128)

    zeros_acc = jnp.zeros((NPAD, D), jnp.float32)

    degp = _deg_call(dst_p, ew_p, zeros_acc)
    deg_col = (degp[0, :N, 0] + degp[1, :N, 0] + 1.0).reshape(N, 1)

    b1r = b1.reshape(1, D)
    b2r = b2.reshape(1, D)

    xs1, t1, dinv_col = _enc_call(x, W1, b1r, deg_col)

    acc1 = _msg_call(xs1, src_p, dst_p, ew_p, zeros_acc)[:, :N, :]
    h1, xs2, t2 = _mid_call(acc1, t1, dinv_col, W2, b2r)
    acc2 = _msg_call(xs2, src_p, dst_p, ew_p, zeros_acc)[:, :N, :]

    batch3 = batch.astype(jnp.int32).reshape(NBLK, 1, BLK)
    wc_pad = jnp.zeros((2 * D, 128), jnp.float32).at[:, :C].set(Wc)
    bc_pad = jnp.zeros((1, 128), jnp.float32).at[:, :C].set(bc.reshape(1, C))

    logits_pad = _fin_call(acc2, t2, dinv_col, h1, batch3, wc_pad, bc_pad)
    return logits_pad[:, :C]
